# baseline (speedup 1.0000x reference)
import jax
import jax.numpy as jnp
from jax.experimental import pallas as pl
from jax.experimental.pallas import tpu as pltpu

_B, _F, _T = 64, 128, 4096
_NDMA = 8


def _body(x_hbm, o_hbm, sem):
    for i in range(_NDMA):
        pltpu.make_async_copy(
            x_hbm.at[pl.ds(i * (_B // _NDMA), _B // _NDMA)],
            o_hbm.at[pl.ds(i * (_B // _NDMA), _B // _NDMA)],
            sem.at[i],
        ).start()
    for i in range(_NDMA):
        pltpu.make_async_copy(
            x_hbm.at[pl.ds(i * (_B // _NDMA), _B // _NDMA)],
            o_hbm.at[pl.ds(i * (_B // _NDMA), _B // _NDMA)],
            sem.at[i],
        ).wait()


def kernel(x, f0, t0):
    return pl.pallas_call(
        _body,
        in_specs=[pl.BlockSpec(memory_space=pltpu.HBM)],
        out_specs=pl.BlockSpec(memory_space=pltpu.HBM),
        scratch_shapes=[pltpu.SemaphoreType.DMA((_NDMA,))],
        out_shape=jax.ShapeDtypeStruct(x.shape, x.dtype),
    )(x)


# speedup vs baseline: 38.7864x; 38.7864x over previous
import jax
import jax.numpy as jnp
from jax.experimental import pallas as pl
from jax.experimental.pallas import tpu as pltpu

_B, _F, _T = 64, 128, 4096
_HS = 64
_TC = 512


def _body(f0_ref, t0_ref, x_ref, o_ref):
    b = pl.program_id(0)
    o_ref[0] = x_ref[0]
    for h in range(2):
        f = f0_ref[b, h]
        t = t0_ref[b, h]
        for rb in range(0, _F, _HS):
            rowhit = (f < rb + _HS) & (f + _HS > rb)
            for c in range(_T // _TC):
                lo = c * _TC

                @pl.when(rowhit & (t < lo + _TC) & (t + _HS > lo))
                def _(rb=rb, lo=lo, f=f, t=t):
                    fi = rb + jax.lax.broadcasted_iota(jnp.int32, (_HS, 1), 0)
                    ti = lo + jax.lax.broadcasted_iota(jnp.int32, (1, _TC), 1)
                    m = ((fi >= f) & (fi < f + _HS)
                         & (ti >= t) & (ti < t + _HS))
                    o_ref[0, rb:rb + _HS, lo:lo + _TC] = jnp.where(
                        m, jnp.zeros((), o_ref.dtype),
                        o_ref[0, rb:rb + _HS, lo:lo + _TC])


def kernel(x, f0, t0):
    grid_spec = pltpu.PrefetchScalarGridSpec(
        num_scalar_prefetch=2,
        grid=(_B,),
        in_specs=[pl.BlockSpec((1, _F, _T), lambda b, *_: (b, 0, 0))],
        out_specs=pl.BlockSpec((1, _F, _T), lambda b, *_: (b, 0, 0)),
    )
    return pl.pallas_call(
        _body,
        grid_spec=grid_spec,
        out_shape=jax.ShapeDtypeStruct(x.shape, x.dtype),
    )(f0.astype(jnp.int32), t0.astype(jnp.int32), x)
